# Initial kernel scaffold; baseline (speedup 1.0000x reference)
import jax
import jax.numpy as jnp
from jax.experimental import pallas as pl


def kernel(x, edge_index, batch, W1, b1, W2, b2):
    raise NotImplementedError("write your pallas kernel here")


# speedup vs baseline: 11.6594x; 11.6594x over previous
import functools

import jax
import jax.numpy as jnp
from jax import lax
from jax.experimental import pallas as pl
from jax.experimental.pallas import tpu as pltpu
from jax.experimental.pallas import tpu_sc as plsc

N = 10000
E = 320000
D = 128
G = 16

NC = 2
NS = 16
NW = NC * NS
NP = 10240
K = 80
EPT = E // NW
ROWS_PER_TILE = NP // NS

def _deg_body(dst_hbm, ones_hbm, zeros_hbm, out_hbm, idx_v, ones_v, hist):
    c = lax.axis_index("c")
    s = lax.axis_index("s")
    r0 = s * ROWS_PER_TILE
    pltpu.sync_copy(zeros_hbm.at[pl.ds(r0, ROWS_PER_TILE)],
                    hist.at[pl.ds(r0, ROWS_PER_TILE)])
    pltpu.sync_copy(ones_hbm, ones_v)
    plsc.subcore_barrier()
    base0 = (c * NS + s) * EPT

    def body(t, carry):
        base = base0 + t * K
        pltpu.sync_copy(dst_hbm.at[pl.ds(base, K)], idx_v)
        pltpu.sync_copy(ones_v, hist.at[idx_v], add=True)
        return carry

    lax.fori_loop(0, EPT // K, body, 0)
    plsc.subcore_barrier()
    pltpu.sync_copy(hist.at[pl.ds(r0, ROWS_PER_TILE)],
                    out_hbm.at[c, pl.ds(r0, ROWS_PER_TILE)])


@functools.cache
def _sc_calls():
    mesh = plsc.VectorSubcoreMesh(core_axis_name="c", subcore_axis_name="s",
                                  num_cores=NC, num_subcores=NS)
    deg_call = pl.kernel(
        _deg_body,
        out_type=jax.ShapeDtypeStruct((NC, NP, D), jnp.float32),
        mesh=mesh,
        scratch_types=[
            pltpu.VMEM((K,), jnp.int32),
            pltpu.VMEM((K, D), jnp.float32),
            pltpu.VMEM_SHARED((NP, D), jnp.float32),
        ],
    )
    agg_call = pl.kernel(
        _agg_body,
        out_type=jax.ShapeDtypeStruct((NC, NP, D), jnp.float32),
        mesh=mesh,
        scratch_types=[
            pltpu.VMEM((K,), jnp.int32),
            pltpu.VMEM((K,), jnp.int32),
            pltpu.VMEM((K, D), jnp.float32),
            pltpu.VMEM_SHARED((NP, D), jnp.float32),
            pltpu.SemaphoreType.DMA,
        ],
    )
    return deg_call, agg_call


def _agg_body(g_hbm, src_hbm, dst_hbm, zeros_hbm, out_hbm,
              src_v, dst_v, rows_v, acc, sem):
    c = lax.axis_index("c")
    s = lax.axis_index("s")
    r0 = s * ROWS_PER_TILE
    pltpu.sync_copy(zeros_hbm.at[pl.ds(r0, ROWS_PER_TILE)],
                    acc.at[pl.ds(r0, ROWS_PER_TILE)])
    plsc.subcore_barrier()
    base0 = (c * NS + s) * EPT

    def body(t, carry):
        base = base0 + t * K
        pltpu.sync_copy(src_hbm.at[pl.ds(base, K)], src_v)
        pltpu.sync_copy(dst_hbm.at[pl.ds(base, K)], dst_v)
        pltpu.async_copy(g_hbm.at[src_v], rows_v, sem).wait()
        pltpu.sync_copy(rows_v, acc.at[dst_v], add=True)
        return carry

    lax.fori_loop(0, EPT // K, body, 0)
    plsc.subcore_barrier()
    pltpu.sync_copy(acc.at[pl.ds(r0, ROWS_PER_TILE)],
                    out_hbm.at[c, pl.ds(r0, ROWS_PER_TILE)])


B = 512


def _dinv(h0_ref, h1_ref):
    deg = h0_ref[:, 0:1] + h1_ref[:, 0:1] + 1.0
    return lax.rsqrt(deg)


def _l1_body(x_ref, w_ref, h0_ref, h1_ref, g_ref):
    h = lax.dot_general(x_ref[...], w_ref[...], (((1,), (1,)), ((), ())),
                        preferred_element_type=jnp.float32)
    g_ref[...] = h * _dinv(h0_ref, h1_ref)


def _l2_body(s0_ref, s1_ref, g1_ref, h0_ref, h1_ref, w_ref, b_ref, g2_ref):
    dinv = _dinv(h0_ref, h1_ref)
    z = dinv * (s0_ref[...] + s1_ref[...] + g1_ref[...]) + b_ref[...]
    z = jnp.maximum(z, 0.0)
    h = lax.dot_general(z, w_ref[...], (((1,), (1,)), ((), ())),
                        preferred_element_type=jnp.float32)
    g2_ref[...] = h * dinv


def _fin_body(s0_ref, s1_ref, g2_ref, h0_ref, h1_ref, b_ref, bat_ref, out_ref):
    i = pl.program_id(0)
    dinv = _dinv(h0_ref, h1_ref)
    y = dinv * (s0_ref[...] + s1_ref[...] + g2_ref[...]) + b_ref[...]
    bat = bat_ref[...]
    neg = jnp.float32(-jnp.inf)

    @pl.when(i == 0)
    def _():
        out_ref[...] = jnp.full((G, D), neg, jnp.float32)

    rows = []
    for g in range(G):
        v = jnp.where(bat == jnp.float32(g), y, neg)
        rows.append(v.max(axis=0, keepdims=True))
    out_ref[...] = jnp.maximum(out_ref[...], jnp.concatenate(rows, axis=0))


_row_spec = pl.BlockSpec((B, D), lambda i: (i, 0))
_hist_spec = pl.BlockSpec((B, D), lambda i: (i, 0))
_w_spec = pl.BlockSpec((D, D), lambda i: (0, 0))
_b_spec = pl.BlockSpec((1, D), lambda i: (0, 0))

_l1_call = pl.pallas_call(
    _l1_body,
    grid=(NP // B,),
    in_specs=[_row_spec, _w_spec, _hist_spec, _hist_spec],
    out_specs=_row_spec,
    out_shape=jax.ShapeDtypeStruct((NP, D), jnp.float32),
)

_l2_call = pl.pallas_call(
    _l2_body,
    grid=(NP // B,),
    in_specs=[_row_spec, _row_spec, _row_spec, _hist_spec, _hist_spec,
              _w_spec, _b_spec],
    out_specs=_row_spec,
    out_shape=jax.ShapeDtypeStruct((NP, D), jnp.float32),
)

_fin_call = pl.pallas_call(
    _fin_body,
    grid=(NP // B,),
    in_specs=[_row_spec, _row_spec, _row_spec, _hist_spec, _hist_spec,
              _b_spec, _row_spec],
    out_specs=pl.BlockSpec((G, D), lambda i: (0, 0)),
    out_shape=jax.ShapeDtypeStruct((G, D), jnp.float32),
)


def kernel(x, edge_index, batch, W1, b1, W2, b2):
    src = edge_index[0]
    dst = edge_index[1]
    x_p = jnp.pad(x, ((0, NP - N), (0, 0)))
    batf = jnp.pad(batch.astype(jnp.float32), (0, NP - N),
                   constant_values=1e9)
    batf = jnp.broadcast_to(batf[:, None], (NP, D))
    zeros128 = jnp.zeros((NP, D), jnp.float32)
    ones = jnp.ones((K, D), jnp.float32)

    _deg_call, _agg_call = _sc_calls()
    hist = _deg_call(dst, ones, zeros128)
    h0, h1 = hist[0], hist[1]
    g1 = _l1_call(x_p, W1, h0, h1)
    s1 = _agg_call(g1, src, dst, zeros128)
    g2 = _l2_call(s1[0], s1[1], g1, h0, h1, W2, b1.reshape(1, D))
    s2 = _agg_call(g2, src, dst, zeros128)
    return _fin_call(s2[0], s2[1], g2, h0, h1, b2.reshape(1, D), batf)


# speedup vs baseline: 18.1155x; 1.5537x over previous
import functools

import jax
import jax.numpy as jnp
from jax import lax
from jax.experimental import pallas as pl
from jax.experimental.pallas import tpu as pltpu
from jax.experimental.pallas import tpu_sc as plsc

N = 10000
E = 320000
D = 128
G = 16

NC = 2
NS = 16
NW = NC * NS
NP = 10240
K = 80
EPT = E // NW
TPT = EPT // K
PAIRS = (TPT - 1) // 2
ROWS_PER_TILE = NP // NS


def _deg_body(dst_hbm, ones_hbm, zeros_hbm, out_hbm,
              dst0, dst1, ones_v, hist, sd0, sd1):
    c = lax.axis_index("c")
    s = lax.axis_index("s")
    r0 = s * ROWS_PER_TILE
    pltpu.sync_copy(zeros_hbm.at[pl.ds(r0, ROWS_PER_TILE)],
                    hist.at[pl.ds(r0, ROWS_PER_TILE)])
    pltpu.sync_copy(ones_hbm, ones_v)
    plsc.subcore_barrier()
    base0 = (c * NS + s) * EPT

    def body(t, carry):
        pltpu.sync_copy(dst_hbm.at[pl.ds(base0 + t * K, K)], dst0)
        pltpu.sync_copy(ones_v, hist.at[dst0], add=True)
        return carry

    lax.fori_loop(0, TPT, body, 0)
    plsc.subcore_barrier()
    pltpu.sync_copy(hist.at[pl.ds(r0, ROWS_PER_TILE)],
                    out_hbm.at[c, pl.ds(r0, ROWS_PER_TILE)])


def _agg_body(g_hbm, src_hbm, dst_hbm, zeros_hbm, out_hbm,
              src0, src1, dst0, dst1, rows0, rows1,
              acc, si0, si1, sg0, sg1):
    c = lax.axis_index("c")
    s = lax.axis_index("s")
    r0 = s * ROWS_PER_TILE
    pltpu.sync_copy(zeros_hbm.at[pl.ds(r0, ROWS_PER_TILE)],
                    acc.at[pl.ds(r0, ROWS_PER_TILE)])
    plsc.subcore_barrier()
    base0 = (c * NS + s) * EPT

    def idx_start(t, srcv, dstv, si):
        base = base0 + t * K
        pltpu.async_copy(src_hbm.at[pl.ds(base, K)], srcv, si)
        pltpu.async_copy(dst_hbm.at[pl.ds(base, K)], dstv, si)

    def idx_wait(t, srcv, dstv, si):
        base = base0 + t * K
        pltpu.make_async_copy(src_hbm.at[pl.ds(base, K)], srcv, si).wait()
        pltpu.make_async_copy(dst_hbm.at[pl.ds(base, K)], dstv, si).wait()

    idx_start(0, src0, dst0, si0)
    idx_start(1, src1, dst1, si1)
    idx_wait(0, src0, dst0, si0)
    pltpu.async_copy(g_hbm.at[src0], rows0, sg0)

    def half(t, srcv, dstv, si, rows, sg, srcn, dstn, sin, rowsn, sgn):
        pltpu.make_async_copy(g_hbm.at[srcv], rows, sg).wait()

        @pl.when(t + 1 < TPT)
        def _():
            idx_wait(t + 1, srcn, dstn, sin)
            pltpu.async_copy(g_hbm.at[srcn], rowsn, sgn)

        pltpu.sync_copy(rows, acc.at[dstv], add=True)

        @pl.when(t + 2 < TPT)
        def _():
            idx_start(t + 2, srcv, dstv, si)

    def body(i, carry):
        t = 2 * i
        half(t, src0, dst0, si0, rows0, sg0, src1, dst1, si1, rows1, sg1)
        half(t + 1, src1, dst1, si1, rows1, sg1, src0, dst0, si0, rows0, sg0)
        return carry

    lax.fori_loop(0, PAIRS, body, 0)
    half(TPT - 1, src0, dst0, si0, rows0, sg0, src1, dst1, si1, rows1, sg1)
    plsc.subcore_barrier()
    pltpu.sync_copy(acc.at[pl.ds(r0, ROWS_PER_TILE)],
                    out_hbm.at[c, pl.ds(r0, ROWS_PER_TILE)])


@functools.cache
def _sc_calls():
    mesh = plsc.VectorSubcoreMesh(core_axis_name="c", subcore_axis_name="s",
                                  num_cores=NC, num_subcores=NS)
    deg_call = pl.kernel(
        _deg_body,
        out_type=jax.ShapeDtypeStruct((NC, NP, D), jnp.float32),
        mesh=mesh,
        scratch_types=[
            pltpu.VMEM((K,), jnp.int32),
            pltpu.VMEM((K,), jnp.int32),
            pltpu.VMEM((K, D), jnp.float32),
            pltpu.VMEM_SHARED((NP, D), jnp.float32),
            pltpu.SemaphoreType.DMA,
            pltpu.SemaphoreType.DMA,
        ],
    )
    agg_call = pl.kernel(
        _agg_body,
        out_type=jax.ShapeDtypeStruct((NC, NP, D), jnp.float32),
        mesh=mesh,
        scratch_types=[
            pltpu.VMEM((K,), jnp.int32),
            pltpu.VMEM((K,), jnp.int32),
            pltpu.VMEM((K,), jnp.int32),
            pltpu.VMEM((K,), jnp.int32),
            pltpu.VMEM((K, D), jnp.float32),
            pltpu.VMEM((K, D), jnp.float32),
            pltpu.VMEM_SHARED((NP, D), jnp.float32),
            pltpu.SemaphoreType.DMA,
            pltpu.SemaphoreType.DMA,
            pltpu.SemaphoreType.DMA,
            pltpu.SemaphoreType.DMA,
        ],
    )
    return deg_call, agg_call


B = 512


def _dinv(h0_ref, h1_ref):
    deg = h0_ref[:, 0:1] + h1_ref[:, 0:1] + 1.0
    return lax.rsqrt(deg)


def _l1_body(x_ref, w_ref, h0_ref, h1_ref, g_ref):
    h = lax.dot_general(x_ref[...], w_ref[...], (((1,), (1,)), ((), ())),
                        preferred_element_type=jnp.float32)
    g_ref[...] = h * _dinv(h0_ref, h1_ref)


def _l2_body(s0_ref, s1_ref, g1_ref, h0_ref, h1_ref, w_ref, b_ref, g2_ref):
    dinv = _dinv(h0_ref, h1_ref)
    z = dinv * (s0_ref[...] + s1_ref[...] + g1_ref[...]) + b_ref[...]
    z = jnp.maximum(z, 0.0)
    h = lax.dot_general(z, w_ref[...], (((1,), (1,)), ((), ())),
                        preferred_element_type=jnp.float32)
    g2_ref[...] = h * dinv


def _fin_body(s0_ref, s1_ref, g2_ref, h0_ref, h1_ref, b_ref, bat_ref, out_ref):
    i = pl.program_id(0)
    dinv = _dinv(h0_ref, h1_ref)
    y = dinv * (s0_ref[...] + s1_ref[...] + g2_ref[...]) + b_ref[...]
    bat = bat_ref[...]
    neg = jnp.float32(-jnp.inf)

    @pl.when(i == 0)
    def _():
        out_ref[...] = jnp.full((G, D), neg, jnp.float32)

    rows = []
    for g in range(G):
        v = jnp.where(bat == jnp.float32(g), y, neg)
        rows.append(v.max(axis=0, keepdims=True))
    out_ref[...] = jnp.maximum(out_ref[...], jnp.concatenate(rows, axis=0))


_row_spec = pl.BlockSpec((B, D), lambda i: (i, 0))
_hist_spec = pl.BlockSpec((B, D), lambda i: (i, 0))
_w_spec = pl.BlockSpec((D, D), lambda i: (0, 0))
_b_spec = pl.BlockSpec((1, D), lambda i: (0, 0))

_l1_call = pl.pallas_call(
    _l1_body,
    grid=(NP // B,),
    in_specs=[_row_spec, _w_spec, _hist_spec, _hist_spec],
    out_specs=_row_spec,
    out_shape=jax.ShapeDtypeStruct((NP, D), jnp.float32),
)

_l2_call = pl.pallas_call(
    _l2_body,
    grid=(NP // B,),
    in_specs=[_row_spec, _row_spec, _row_spec, _hist_spec, _hist_spec,
              _w_spec, _b_spec],
    out_specs=_row_spec,
    out_shape=jax.ShapeDtypeStruct((NP, D), jnp.float32),
)

_fin_call = pl.pallas_call(
    _fin_body,
    grid=(NP // B,),
    in_specs=[_row_spec, _row_spec, _row_spec, _hist_spec, _hist_spec,
              _b_spec, _row_spec],
    out_specs=pl.BlockSpec((G, D), lambda i: (0, 0)),
    out_shape=jax.ShapeDtypeStruct((G, D), jnp.float32),
)


def kernel(x, edge_index, batch, W1, b1, W2, b2):
    src2 = edge_index[0]
    dst2 = edge_index[1]
    x_p = jnp.pad(x, ((0, NP - N), (0, 0)))
    batf = jnp.pad(batch.astype(jnp.float32), (0, NP - N),
                   constant_values=1e9)
    batf = jnp.broadcast_to(batf[:, None], (NP, D))
    zeros128 = jnp.zeros((NP, D), jnp.float32)
    ones = jnp.ones((K, D), jnp.float32)

    _deg_call, _agg_call = _sc_calls()
    hist = _deg_call(dst2, ones, zeros128)
    h0, h1 = hist[0], hist[1]
    g1 = _l1_call(x_p, W1, h0, h1)
    s1 = _agg_call(g1, src2, dst2, zeros128)
    g2 = _l2_call(s1[0], s1[1], g1, h0, h1, W2, b1.reshape(1, D))
    s2 = _agg_call(g2, src2, dst2, zeros128)
    return _fin_call(s2[0], s2[1], g2, h0, h1, b2.reshape(1, D), batf)


# speedup vs baseline: 19.9511x; 1.1013x over previous
import functools

import jax
import jax.numpy as jnp
from jax import lax
from jax.experimental import pallas as pl
from jax.experimental.pallas import tpu as pltpu
from jax.experimental.pallas import tpu_sc as plsc

N = 10000
E = 320000
D = 128
G = 16

NC = 2
NS = 16
NW = NC * NS
NP = 10240
K = 80
EPT = E // NW
TPT = EPT // K
PAIRS = (TPT - 1) // 2
ROWS_PER_TILE = NP // NS


def _deg_body(dst_hbm, ones_hbm, zeros_hbm, out_hbm,
              dst0, dst1, ones_v, hist, sd0, sd1):
    c = lax.axis_index("c")
    s = lax.axis_index("s")
    r0 = s * ROWS_PER_TILE
    pltpu.sync_copy(zeros_hbm.at[pl.ds(r0, ROWS_PER_TILE)],
                    hist.at[pl.ds(r0, ROWS_PER_TILE)])
    pltpu.sync_copy(ones_hbm, ones_v)
    plsc.subcore_barrier()
    base0 = (c * NS + s) * EPT

    def idx_start(t, v, si):
        pltpu.async_copy(dst_hbm.at[pl.ds(base0 + t * K, K)], v, si)

    idx_start(0, dst0, sd0)
    idx_start(1, dst1, sd1)

    def half(t, v, si):
        pltpu.make_async_copy(dst_hbm.at[pl.ds(base0 + t * K, K)], v, si).wait()
        pltpu.sync_copy(ones_v, hist.at[v], add=True)

        @pl.when(t + 2 < TPT)
        def _():
            idx_start(t + 2, v, si)

    def body(i, carry):
        half(2 * i, dst0, sd0)
        half(2 * i + 1, dst1, sd1)
        return carry

    lax.fori_loop(0, PAIRS, body, 0)
    half(TPT - 1, dst0, sd0)
    plsc.subcore_barrier()
    pltpu.sync_copy(hist.at[pl.ds(r0, ROWS_PER_TILE)],
                    out_hbm.at[c, pl.ds(r0, ROWS_PER_TILE)])


def _agg_body(g_hbm, src_hbm, dst_hbm, zeros_hbm, out_hbm,
              src0, src1, dst0, dst1, rows0, rows1,
              acc, si0, si1, sg0, sg1):
    c = lax.axis_index("c")
    s = lax.axis_index("s")
    r0 = s * ROWS_PER_TILE
    pltpu.sync_copy(zeros_hbm.at[pl.ds(r0, ROWS_PER_TILE)],
                    acc.at[pl.ds(r0, ROWS_PER_TILE)])
    plsc.subcore_barrier()
    base0 = (c * NS + s) * EPT

    def idx_start(t, srcv, dstv, si):
        base = base0 + t * K
        pltpu.async_copy(src_hbm.at[pl.ds(base, K)], srcv, si)
        pltpu.async_copy(dst_hbm.at[pl.ds(base, K)], dstv, si)

    def idx_wait(t, srcv, dstv, si):
        base = base0 + t * K
        pltpu.make_async_copy(src_hbm.at[pl.ds(base, K)], srcv, si).wait()
        pltpu.make_async_copy(dst_hbm.at[pl.ds(base, K)], dstv, si).wait()

    idx_start(0, src0, dst0, si0)
    idx_start(1, src1, dst1, si1)
    idx_wait(0, src0, dst0, si0)
    pltpu.async_copy(g_hbm.at[src0], rows0, sg0)

    def half(t, srcv, dstv, si, rows, sg, srcn, dstn, sin, rowsn, sgn):
        pltpu.make_async_copy(g_hbm.at[srcv], rows, sg).wait()

        @pl.when(t + 1 < TPT)
        def _():
            idx_wait(t + 1, srcn, dstn, sin)
            pltpu.async_copy(g_hbm.at[srcn], rowsn, sgn)

        pltpu.sync_copy(rows, acc.at[dstv], add=True)

        @pl.when(t + 2 < TPT)
        def _():
            idx_start(t + 2, srcv, dstv, si)

    def body(i, carry):
        t = 2 * i
        half(t, src0, dst0, si0, rows0, sg0, src1, dst1, si1, rows1, sg1)
        half(t + 1, src1, dst1, si1, rows1, sg1, src0, dst0, si0, rows0, sg0)
        return carry

    lax.fori_loop(0, PAIRS, body, 0)
    half(TPT - 1, src0, dst0, si0, rows0, sg0, src1, dst1, si1, rows1, sg1)
    plsc.subcore_barrier()
    pltpu.sync_copy(acc.at[pl.ds(r0, ROWS_PER_TILE)],
                    out_hbm.at[c, pl.ds(r0, ROWS_PER_TILE)])


@functools.cache
def _sc_calls():
    mesh = plsc.VectorSubcoreMesh(core_axis_name="c", subcore_axis_name="s",
                                  num_cores=NC, num_subcores=NS)
    deg_call = pl.kernel(
        _deg_body,
        out_type=jax.ShapeDtypeStruct((NC, NP, D), jnp.float32),
        mesh=mesh,
        scratch_types=[
            pltpu.VMEM((K,), jnp.int32),
            pltpu.VMEM((K,), jnp.int32),
            pltpu.VMEM((K, D), jnp.float32),
            pltpu.VMEM_SHARED((NP, D), jnp.float32),
            pltpu.SemaphoreType.DMA,
            pltpu.SemaphoreType.DMA,
        ],
    )
    agg_call = pl.kernel(
        _agg_body,
        out_type=jax.ShapeDtypeStruct((NC, NP, D), jnp.float32),
        mesh=mesh,
        scratch_types=[
            pltpu.VMEM((K,), jnp.int32),
            pltpu.VMEM((K,), jnp.int32),
            pltpu.VMEM((K,), jnp.int32),
            pltpu.VMEM((K,), jnp.int32),
            pltpu.VMEM((K, D), jnp.float32),
            pltpu.VMEM((K, D), jnp.float32),
            pltpu.VMEM_SHARED((NP, D), jnp.float32),
            pltpu.SemaphoreType.DMA,
            pltpu.SemaphoreType.DMA,
            pltpu.SemaphoreType.DMA,
            pltpu.SemaphoreType.DMA,
        ],
    )
    return deg_call, agg_call


B = 512


def _dinv(h0_ref, h1_ref):
    deg = h0_ref[:, 0:1] + h1_ref[:, 0:1] + 1.0
    return lax.rsqrt(deg)


def _l1_body(x_ref, w_ref, h0_ref, h1_ref, g_ref):
    h = lax.dot_general(x_ref[...], w_ref[...], (((1,), (1,)), ((), ())),
                        preferred_element_type=jnp.float32)
    g_ref[...] = h * _dinv(h0_ref, h1_ref)


def _l2_body(s0_ref, s1_ref, g1_ref, h0_ref, h1_ref, w_ref, b_ref, g2_ref):
    dinv = _dinv(h0_ref, h1_ref)
    z = dinv * (s0_ref[...] + s1_ref[...] + g1_ref[...]) + b_ref[...]
    z = jnp.maximum(z, 0.0)
    h = lax.dot_general(z, w_ref[...], (((1,), (1,)), ((), ())),
                        preferred_element_type=jnp.float32)
    g2_ref[...] = h * dinv


def _fin_body(s0_ref, s1_ref, g2_ref, h0_ref, h1_ref, b_ref, bat_ref, out_ref):
    i = pl.program_id(0)
    dinv = _dinv(h0_ref, h1_ref)
    y = dinv * (s0_ref[...] + s1_ref[...] + g2_ref[...]) + b_ref[...]
    bat = bat_ref[...]
    neg = jnp.float32(-jnp.inf)

    @pl.when(i == 0)
    def _():
        out_ref[...] = jnp.full((G, D), neg, jnp.float32)

    rows = []
    for g in range(G):
        v = jnp.where(bat == jnp.float32(g), y, neg)
        rows.append(v.max(axis=0, keepdims=True))
    out_ref[...] = jnp.maximum(out_ref[...], jnp.concatenate(rows, axis=0))


_row_spec = pl.BlockSpec((B, D), lambda i: (i, 0))
_hist_spec = pl.BlockSpec((B, D), lambda i: (i, 0))
_w_spec = pl.BlockSpec((D, D), lambda i: (0, 0))
_b_spec = pl.BlockSpec((1, D), lambda i: (0, 0))

_l1_call = pl.pallas_call(
    _l1_body,
    grid=(NP // B,),
    in_specs=[_row_spec, _w_spec, _hist_spec, _hist_spec],
    out_specs=_row_spec,
    out_shape=jax.ShapeDtypeStruct((NP, D), jnp.float32),
)

_l2_call = pl.pallas_call(
    _l2_body,
    grid=(NP // B,),
    in_specs=[_row_spec, _row_spec, _row_spec, _hist_spec, _hist_spec,
              _w_spec, _b_spec],
    out_specs=_row_spec,
    out_shape=jax.ShapeDtypeStruct((NP, D), jnp.float32),
)

_fin_call = pl.pallas_call(
    _fin_body,
    grid=(NP // B,),
    in_specs=[_row_spec, _row_spec, _row_spec, _hist_spec, _hist_spec,
              _b_spec, _row_spec],
    out_specs=pl.BlockSpec((G, D), lambda i: (0, 0)),
    out_shape=jax.ShapeDtypeStruct((G, D), jnp.float32),
)


def kernel(x, edge_index, batch, W1, b1, W2, b2):
    src2 = edge_index[0]
    dst2 = edge_index[1]
    x_p = jnp.pad(x, ((0, NP - N), (0, 0)))
    batf = jnp.pad(batch.astype(jnp.float32), (0, NP - N),
                   constant_values=1e9)
    batf = jnp.broadcast_to(batf[:, None], (NP, D))
    zeros128 = jnp.zeros((NP, D), jnp.float32)
    ones = jnp.ones((K, D), jnp.float32)

    _deg_call, _agg_call = _sc_calls()
    hist = _deg_call(dst2, ones, zeros128)
    h0, h1 = hist[0], hist[1]
    g1 = _l1_call(x_p, W1, h0, h1)
    s1 = _agg_call(g1, src2, dst2, zeros128)
    g2 = _l2_call(s1[0], s1[1], g1, h0, h1, W2, b1.reshape(1, D))
    s2 = _agg_call(g2, src2, dst2, zeros128)
    return _fin_call(s2[0], s2[1], g2, h0, h1, b2.reshape(1, D), batf)


# speedup vs baseline: 21.7676x; 1.0910x over previous
import functools

import jax
import jax.numpy as jnp
from jax import lax
from jax.experimental import pallas as pl
from jax.experimental.pallas import tpu as pltpu
from jax.experimental.pallas import tpu_sc as plsc

N = 10000
E = 320000
D = 128
G = 16

NC = 2
NS = 16
NW = NC * NS
NP = 10240
K = 80
EPT = E // NW
TPT = EPT // K
PAIRS = (TPT - 1) // 2
TRIPLES = (TPT - 2) // 3
ROWS_PER_TILE = NP // NS


def _deg_body(dst_hbm, ones_hbm, zeros_hbm, out_hbm,
              dst0, dst1, ones_v, hist, sd0, sd1):
    c = lax.axis_index("c")
    s = lax.axis_index("s")
    r0 = s * ROWS_PER_TILE
    pltpu.sync_copy(zeros_hbm.at[pl.ds(r0, ROWS_PER_TILE)],
                    hist.at[pl.ds(r0, ROWS_PER_TILE)])
    pltpu.sync_copy(ones_hbm, ones_v)
    plsc.subcore_barrier()
    base0 = (c * NS + s) * EPT

    def idx_start(t, v, si):
        pltpu.async_copy(dst_hbm.at[pl.ds(base0 + t * K, K)], v, si)

    idx_start(0, dst0, sd0)
    idx_start(1, dst1, sd1)

    def half(t, v, si):
        pltpu.make_async_copy(dst_hbm.at[pl.ds(base0 + t * K, K)], v, si).wait()
        pltpu.sync_copy(ones_v, hist.at[v], add=True)

        @pl.when(t + 2 < TPT)
        def _():
            idx_start(t + 2, v, si)

    def body(i, carry):
        half(2 * i, dst0, sd0)
        half(2 * i + 1, dst1, sd1)
        return carry

    lax.fori_loop(0, PAIRS, body, 0)
    half(TPT - 1, dst0, sd0)
    plsc.subcore_barrier()
    pltpu.sync_copy(hist.at[pl.ds(r0, ROWS_PER_TILE)],
                    out_hbm.at[c, pl.ds(r0, ROWS_PER_TILE)])


def _agg_body(g_hbm, src_hbm, dst_hbm, zeros_hbm, out_hbm,
              src0, src1, src2, dst0, dst1, dst2, rows0, rows1, rows2,
              acc, si0, si1, si2, sg0, sg1, sg2):
    c = lax.axis_index("c")
    s = lax.axis_index("s")
    r0 = s * ROWS_PER_TILE
    pltpu.sync_copy(zeros_hbm.at[pl.ds(r0, ROWS_PER_TILE)],
                    acc.at[pl.ds(r0, ROWS_PER_TILE)])
    plsc.subcore_barrier()
    base0 = (c * NS + s) * EPT

    def idx_start(t, srcv, dstv, si):
        base = base0 + t * K
        pltpu.async_copy(src_hbm.at[pl.ds(base, K)], srcv, si)
        pltpu.async_copy(dst_hbm.at[pl.ds(base, K)], dstv, si)

    def idx_wait(t, srcv, dstv, si):
        base = base0 + t * K
        pltpu.make_async_copy(src_hbm.at[pl.ds(base, K)], srcv, si).wait()
        pltpu.make_async_copy(dst_hbm.at[pl.ds(base, K)], dstv, si).wait()

    idx_start(0, src0, dst0, si0)
    idx_start(1, src1, dst1, si1)
    idx_start(2, src2, dst2, si2)
    idx_wait(0, src0, dst0, si0)
    pltpu.async_copy(g_hbm.at[src0], rows0, sg0)
    idx_wait(1, src1, dst1, si1)
    pltpu.async_copy(g_hbm.at[src1], rows1, sg1)

    def step(t, srcv, dstv, si, rows, sg, src2n, dst2n, si2n, rows2n, sg2n):
        pltpu.make_async_copy(g_hbm.at[srcv], rows, sg).wait()

        @pl.when(t + 2 < TPT)
        def _():
            idx_wait(t + 2, src2n, dst2n, si2n)
            pltpu.async_copy(g_hbm.at[src2n], rows2n, sg2n)

        pltpu.sync_copy(rows, acc.at[dstv], add=True)

        @pl.when(t + 3 < TPT)
        def _():
            idx_start(t + 3, srcv, dstv, si)

    def body(i, carry):
        t = 3 * i
        step(t, src0, dst0, si0, rows0, sg0, src2, dst2, si2, rows2, sg2)
        step(t + 1, src1, dst1, si1, rows1, sg1, src0, dst0, si0, rows0, sg0)
        step(t + 2, src2, dst2, si2, rows2, sg2, src1, dst1, si1, rows1, sg1)
        return carry

    lax.fori_loop(0, TRIPLES, body, 0)
    step(TPT - 2, src0, dst0, si0, rows0, sg0, src2, dst2, si2, rows2, sg2)
    step(TPT - 1, src1, dst1, si1, rows1, sg1, src0, dst0, si0, rows0, sg0)
    plsc.subcore_barrier()
    pltpu.sync_copy(acc.at[pl.ds(r0, ROWS_PER_TILE)],
                    out_hbm.at[c, pl.ds(r0, ROWS_PER_TILE)])


@functools.cache
def _sc_calls():
    mesh = plsc.VectorSubcoreMesh(core_axis_name="c", subcore_axis_name="s",
                                  num_cores=NC, num_subcores=NS)
    deg_call = pl.kernel(
        _deg_body,
        out_type=jax.ShapeDtypeStruct((NC, NP, D), jnp.float32),
        mesh=mesh,
        scratch_types=[
            pltpu.VMEM((K,), jnp.int32),
            pltpu.VMEM((K,), jnp.int32),
            pltpu.VMEM((K, D), jnp.float32),
            pltpu.VMEM_SHARED((NP, D), jnp.float32),
            pltpu.SemaphoreType.DMA,
            pltpu.SemaphoreType.DMA,
        ],
    )
    agg_call = pl.kernel(
        _agg_body,
        out_type=jax.ShapeDtypeStruct((NC, NP, D), jnp.float32),
        mesh=mesh,
        scratch_types=[
            pltpu.VMEM((K,), jnp.int32),
            pltpu.VMEM((K,), jnp.int32),
            pltpu.VMEM((K,), jnp.int32),
            pltpu.VMEM((K,), jnp.int32),
            pltpu.VMEM((K,), jnp.int32),
            pltpu.VMEM((K,), jnp.int32),
            pltpu.VMEM((K, D), jnp.float32),
            pltpu.VMEM((K, D), jnp.float32),
            pltpu.VMEM((K, D), jnp.float32),
            pltpu.VMEM_SHARED((NP, D), jnp.float32),
            pltpu.SemaphoreType.DMA,
            pltpu.SemaphoreType.DMA,
            pltpu.SemaphoreType.DMA,
            pltpu.SemaphoreType.DMA,
            pltpu.SemaphoreType.DMA,
            pltpu.SemaphoreType.DMA,
        ],
    )
    return deg_call, agg_call


B = 512


def _dinv(h0_ref, h1_ref):
    deg = h0_ref[:, 0:1] + h1_ref[:, 0:1] + 1.0
    return lax.rsqrt(deg)


def _l1_body(x_ref, w_ref, h0_ref, h1_ref, g_ref):
    h = lax.dot_general(x_ref[...], w_ref[...], (((1,), (1,)), ((), ())),
                        preferred_element_type=jnp.float32)
    g_ref[...] = h * _dinv(h0_ref, h1_ref)


def _l2_body(s0_ref, s1_ref, g1_ref, h0_ref, h1_ref, w_ref, b_ref, g2_ref):
    dinv = _dinv(h0_ref, h1_ref)
    z = dinv * (s0_ref[...] + s1_ref[...] + g1_ref[...]) + b_ref[...]
    z = jnp.maximum(z, 0.0)
    h = lax.dot_general(z, w_ref[...], (((1,), (1,)), ((), ())),
                        preferred_element_type=jnp.float32)
    g2_ref[...] = h * dinv


def _fin_body(s0_ref, s1_ref, g2_ref, h0_ref, h1_ref, b_ref, bat_ref, out_ref):
    i = pl.program_id(0)
    dinv = _dinv(h0_ref, h1_ref)
    y = dinv * (s0_ref[...] + s1_ref[...] + g2_ref[...]) + b_ref[...]
    bat = bat_ref[...]
    neg = jnp.float32(-jnp.inf)

    @pl.when(i == 0)
    def _():
        out_ref[...] = jnp.full((G, D), neg, jnp.float32)

    rows = []
    for g in range(G):
        v = jnp.where(bat == jnp.float32(g), y, neg)
        rows.append(v.max(axis=0, keepdims=True))
    out_ref[...] = jnp.maximum(out_ref[...], jnp.concatenate(rows, axis=0))


_row_spec = pl.BlockSpec((B, D), lambda i: (i, 0))
_hist_spec = pl.BlockSpec((B, D), lambda i: (i, 0))
_w_spec = pl.BlockSpec((D, D), lambda i: (0, 0))
_b_spec = pl.BlockSpec((1, D), lambda i: (0, 0))

_l1_call = pl.pallas_call(
    _l1_body,
    grid=(NP // B,),
    in_specs=[_row_spec, _w_spec, _hist_spec, _hist_spec],
    out_specs=_row_spec,
    out_shape=jax.ShapeDtypeStruct((NP, D), jnp.float32),
)

_l2_call = pl.pallas_call(
    _l2_body,
    grid=(NP // B,),
    in_specs=[_row_spec, _row_spec, _row_spec, _hist_spec, _hist_spec,
              _w_spec, _b_spec],
    out_specs=_row_spec,
    out_shape=jax.ShapeDtypeStruct((NP, D), jnp.float32),
)

_fin_call = pl.pallas_call(
    _fin_body,
    grid=(NP // B,),
    in_specs=[_row_spec, _row_spec, _row_spec, _hist_spec, _hist_spec,
              _b_spec, _row_spec],
    out_specs=pl.BlockSpec((G, D), lambda i: (0, 0)),
    out_shape=jax.ShapeDtypeStruct((G, D), jnp.float32),
)


def kernel(x, edge_index, batch, W1, b1, W2, b2):
    src2 = edge_index[0]
    dst2 = edge_index[1]
    x_p = jnp.pad(x, ((0, NP - N), (0, 0)))
    batf = jnp.pad(batch.astype(jnp.float32), (0, NP - N),
                   constant_values=1e9)
    batf = jnp.broadcast_to(batf[:, None], (NP, D))
    zeros128 = jnp.zeros((NP, D), jnp.float32)
    ones = jnp.ones((K, D), jnp.float32)

    _deg_call, _agg_call = _sc_calls()
    hist = _deg_call(dst2, ones, zeros128)
    h0, h1 = hist[0], hist[1]
    g1 = _l1_call(x_p, W1, h0, h1)
    s1 = _agg_call(g1, src2, dst2, zeros128)
    g2 = _l2_call(s1[0], s1[1], g1, h0, h1, W2, b1.reshape(1, D))
    s2 = _agg_call(g2, src2, dst2, zeros128)
    return _fin_call(s2[0], s2[1], g2, h0, h1, b2.reshape(1, D), batf)


# speedup vs baseline: 21.7856x; 1.0008x over previous
import functools

import jax
import jax.numpy as jnp
from jax import lax
from jax.experimental import pallas as pl
from jax.experimental.pallas import tpu as pltpu
from jax.experimental.pallas import tpu_sc as plsc

N = 10000
E = 320000
D = 128
G = 16

NC = 2
NS = 16
NW = NC * NS
NP = 10240
K = 80
EPT = E // NW
TPT = EPT // K
PAIRS = (TPT - 1) // 2
QUADS = (TPT - 5) // 4
ROWS_PER_TILE = NP // NS


def _deg_body(dst_hbm, ones_hbm, zeros_hbm, out_hbm,
              dst0, dst1, ones_v, hist, sd0, sd1):
    c = lax.axis_index("c")
    s = lax.axis_index("s")
    r0 = s * ROWS_PER_TILE
    pltpu.sync_copy(zeros_hbm.at[pl.ds(r0, ROWS_PER_TILE)],
                    hist.at[pl.ds(r0, ROWS_PER_TILE)])
    pltpu.sync_copy(ones_hbm, ones_v)
    plsc.subcore_barrier()
    base0 = (c * NS + s) * EPT

    def idx_start(t, v, si):
        pltpu.async_copy(dst_hbm.at[pl.ds(base0 + t * K, K)], v, si)

    idx_start(0, dst0, sd0)
    idx_start(1, dst1, sd1)

    def half(t, v, si):
        pltpu.make_async_copy(dst_hbm.at[pl.ds(base0 + t * K, K)], v, si).wait()
        pltpu.sync_copy(ones_v, hist.at[v], add=True)

        @pl.when(t + 2 < TPT)
        def _():
            idx_start(t + 2, v, si)

    def body(i, carry):
        half(2 * i, dst0, sd0)
        half(2 * i + 1, dst1, sd1)
        return carry

    lax.fori_loop(0, PAIRS, body, 0)
    half(TPT - 1, dst0, sd0)
    plsc.subcore_barrier()
    pltpu.sync_copy(hist.at[pl.ds(r0, ROWS_PER_TILE)],
                    out_hbm.at[c, pl.ds(r0, ROWS_PER_TILE)])


def _agg_body(g_hbm, src_hbm, dst_hbm, zeros_hbm, out_hbm,
              src0, src1, src2, src3, dst0, dst1, dst2, dst3,
              rows0, rows1, rows2, rows3,
              acc, si0, si1, si2, si3, sg0, sg1, sg2, sg3):
    c = lax.axis_index("c")
    s = lax.axis_index("s")
    r0 = s * ROWS_PER_TILE
    pltpu.sync_copy(zeros_hbm.at[pl.ds(r0, ROWS_PER_TILE)],
                    acc.at[pl.ds(r0, ROWS_PER_TILE)])
    plsc.subcore_barrier()
    base0 = (c * NS + s) * EPT

    def idx_start(t, srcv, dstv, si):
        base = base0 + t * K
        pltpu.async_copy(src_hbm.at[pl.ds(base, K)], srcv, si)
        pltpu.async_copy(dst_hbm.at[pl.ds(base, K)], dstv, si)

    def idx_wait(t, srcv, dstv, si):
        base = base0 + t * K
        pltpu.make_async_copy(src_hbm.at[pl.ds(base, K)], srcv, si).wait()
        pltpu.make_async_copy(dst_hbm.at[pl.ds(base, K)], dstv, si).wait()

    idx_start(0, src0, dst0, si0)
    idx_start(1, src1, dst1, si1)
    idx_start(2, src2, dst2, si2)
    idx_start(3, src3, dst3, si3)
    idx_wait(0, src0, dst0, si0)
    pltpu.async_copy(g_hbm.at[src0], rows0, sg0)
    idx_wait(1, src1, dst1, si1)
    pltpu.async_copy(g_hbm.at[src1], rows1, sg1)
    idx_wait(2, src2, dst2, si2)
    pltpu.async_copy(g_hbm.at[src2], rows2, sg2)

    def step(t, srcv, dstv, si, rows, sg, src3n, dst3n, si3n, rows3n, sg3n):
        pltpu.make_async_copy(g_hbm.at[srcv], rows, sg).wait()

        @pl.when(t + 3 < TPT)
        def _():
            idx_wait(t + 3, src3n, dst3n, si3n)
            pltpu.async_copy(g_hbm.at[src3n], rows3n, sg3n)

        pltpu.sync_copy(rows, acc.at[dstv], add=True)

        @pl.when(t + 4 < TPT)
        def _():
            idx_start(t + 4, srcv, dstv, si)

    def body(i, carry):
        t = 4 * i
        step(t, src0, dst0, si0, rows0, sg0, src3, dst3, si3, rows3, sg3)
        step(t + 1, src1, dst1, si1, rows1, sg1, src0, dst0, si0, rows0, sg0)
        step(t + 2, src2, dst2, si2, rows2, sg2, src1, dst1, si1, rows1, sg1)
        step(t + 3, src3, dst3, si3, rows3, sg3, src2, dst2, si2, rows2, sg2)
        return carry

    lax.fori_loop(0, QUADS, body, 0)
    step(TPT - 5, src0, dst0, si0, rows0, sg0, src3, dst3, si3, rows3, sg3)
    step(TPT - 4, src1, dst1, si1, rows1, sg1, src0, dst0, si0, rows0, sg0)
    step(TPT - 3, src2, dst2, si2, rows2, sg2, src1, dst1, si1, rows1, sg1)
    step(TPT - 2, src3, dst3, si3, rows3, sg3, src2, dst2, si2, rows2, sg2)
    step(TPT - 1, src0, dst0, si0, rows0, sg0, src3, dst3, si3, rows3, sg3)
    plsc.subcore_barrier()
    pltpu.sync_copy(acc.at[pl.ds(r0, ROWS_PER_TILE)],
                    out_hbm.at[c, pl.ds(r0, ROWS_PER_TILE)])


@functools.cache
def _sc_calls():
    mesh = plsc.VectorSubcoreMesh(core_axis_name="c", subcore_axis_name="s",
                                  num_cores=NC, num_subcores=NS)
    deg_call = pl.kernel(
        _deg_body,
        out_type=jax.ShapeDtypeStruct((NC, NP, D), jnp.float32),
        mesh=mesh,
        scratch_types=[
            pltpu.VMEM((K,), jnp.int32),
            pltpu.VMEM((K,), jnp.int32),
            pltpu.VMEM((K, D), jnp.float32),
            pltpu.VMEM_SHARED((NP, D), jnp.float32),
            pltpu.SemaphoreType.DMA,
            pltpu.SemaphoreType.DMA,
        ],
    )
    agg_call = pl.kernel(
        _agg_body,
        out_type=jax.ShapeDtypeStruct((NC, NP, D), jnp.float32),
        mesh=mesh,
        scratch_types=[
            pltpu.VMEM((K,), jnp.int32),
            pltpu.VMEM((K,), jnp.int32),
            pltpu.VMEM((K,), jnp.int32),
            pltpu.VMEM((K,), jnp.int32),
            pltpu.VMEM((K,), jnp.int32),
            pltpu.VMEM((K,), jnp.int32),
            pltpu.VMEM((K,), jnp.int32),
            pltpu.VMEM((K,), jnp.int32),
            pltpu.VMEM((K, D), jnp.float32),
            pltpu.VMEM((K, D), jnp.float32),
            pltpu.VMEM((K, D), jnp.float32),
            pltpu.VMEM((K, D), jnp.float32),
            pltpu.VMEM_SHARED((NP, D), jnp.float32),
            pltpu.SemaphoreType.DMA,
            pltpu.SemaphoreType.DMA,
            pltpu.SemaphoreType.DMA,
            pltpu.SemaphoreType.DMA,
            pltpu.SemaphoreType.DMA,
            pltpu.SemaphoreType.DMA,
            pltpu.SemaphoreType.DMA,
            pltpu.SemaphoreType.DMA,
        ],
    )
    return deg_call, agg_call


B = 512


def _dinv(h0_ref, h1_ref):
    deg = h0_ref[:, 0:1] + h1_ref[:, 0:1] + 1.0
    return lax.rsqrt(deg)


def _l1_body(x_ref, w_ref, h0_ref, h1_ref, g_ref):
    h = lax.dot_general(x_ref[...], w_ref[...], (((1,), (1,)), ((), ())),
                        preferred_element_type=jnp.float32)
    g_ref[...] = h * _dinv(h0_ref, h1_ref)


def _l2_body(s0_ref, s1_ref, g1_ref, h0_ref, h1_ref, w_ref, b_ref, g2_ref):
    dinv = _dinv(h0_ref, h1_ref)
    z = dinv * (s0_ref[...] + s1_ref[...] + g1_ref[...]) + b_ref[...]
    z = jnp.maximum(z, 0.0)
    h = lax.dot_general(z, w_ref[...], (((1,), (1,)), ((), ())),
                        preferred_element_type=jnp.float32)
    g2_ref[...] = h * dinv


def _fin_body(s0_ref, s1_ref, g2_ref, h0_ref, h1_ref, b_ref, bat_ref, out_ref):
    i = pl.program_id(0)
    dinv = _dinv(h0_ref, h1_ref)
    y = dinv * (s0_ref[...] + s1_ref[...] + g2_ref[...]) + b_ref[...]
    bat = bat_ref[...]
    neg = jnp.float32(-jnp.inf)

    @pl.when(i == 0)
    def _():
        out_ref[...] = jnp.full((G, D), neg, jnp.float32)

    rows = []
    for g in range(G):
        v = jnp.where(bat == jnp.float32(g), y, neg)
        rows.append(v.max(axis=0, keepdims=True))
    out_ref[...] = jnp.maximum(out_ref[...], jnp.concatenate(rows, axis=0))


_row_spec = pl.BlockSpec((B, D), lambda i: (i, 0))
_hist_spec = pl.BlockSpec((B, D), lambda i: (i, 0))
_w_spec = pl.BlockSpec((D, D), lambda i: (0, 0))
_b_spec = pl.BlockSpec((1, D), lambda i: (0, 0))

_l1_call = pl.pallas_call(
    _l1_body,
    grid=(NP // B,),
    in_specs=[_row_spec, _w_spec, _hist_spec, _hist_spec],
    out_specs=_row_spec,
    out_shape=jax.ShapeDtypeStruct((NP, D), jnp.float32),
)

_l2_call = pl.pallas_call(
    _l2_body,
    grid=(NP // B,),
    in_specs=[_row_spec, _row_spec, _row_spec, _hist_spec, _hist_spec,
              _w_spec, _b_spec],
    out_specs=_row_spec,
    out_shape=jax.ShapeDtypeStruct((NP, D), jnp.float32),
)

_fin_call = pl.pallas_call(
    _fin_body,
    grid=(NP // B,),
    in_specs=[_row_spec, _row_spec, _row_spec, _hist_spec, _hist_spec,
              _b_spec, _row_spec],
    out_specs=pl.BlockSpec((G, D), lambda i: (0, 0)),
    out_shape=jax.ShapeDtypeStruct((G, D), jnp.float32),
)


def kernel(x, edge_index, batch, W1, b1, W2, b2):
    src2 = edge_index[0]
    dst2 = edge_index[1]
    x_p = jnp.pad(x, ((0, NP - N), (0, 0)))
    batf = jnp.pad(batch.astype(jnp.float32), (0, NP - N),
                   constant_values=1e9)
    batf = jnp.broadcast_to(batf[:, None], (NP, D))
    zeros128 = jnp.zeros((NP, D), jnp.float32)
    ones = jnp.ones((K, D), jnp.float32)

    _deg_call, _agg_call = _sc_calls()
    hist = _deg_call(dst2, ones, zeros128)
    h0, h1 = hist[0], hist[1]
    g1 = _l1_call(x_p, W1, h0, h1)
    s1 = _agg_call(g1, src2, dst2, zeros128)
    g2 = _l2_call(s1[0], s1[1], g1, h0, h1, W2, b1.reshape(1, D))
    s2 = _agg_call(g2, src2, dst2, zeros128)
    return _fin_call(s2[0], s2[1], g2, h0, h1, b2.reshape(1, D), batf)
